# Initial kernel scaffold; baseline (speedup 1.0000x reference)
import jax
import jax.numpy as jnp
from jax.experimental import pallas as pl


def kernel(input2, input3, w1, b1, wu1, bu1, wu2, bu2, w2, b2):
    raise NotImplementedError("write your pallas kernel here")


# speedup vs baseline: 1.2474x; 1.2474x over previous
import jax
import jax.numpy as jnp
from jax.experimental import pallas as pl
from jax.experimental.pallas import tpu as pltpu


_K0 = {(0, 0): 1, (0, 1): 2, (1, 0): 0, (1, 1): 1}
_GPOS = {(0, 0): 0, (0, 1): 1, (1, 1): 2, (1, 0): 3}


def _make_body(H, W, Ci, Co):
    Hp, Wp = H + 2, W + 2
    HpWp = Hp * Wp
    HWp = H * Wp
    Cm = 2 * Co + 32
    f32 = jnp.float32
    bf16 = jnp.bfloat16

    def body(x2_ref, x3_ref, mask_ref,
             w1s_ref, wu1s_ref, wu2s_ref, wa_ref, wb_ref, wc_ref, wcn_ref,
             b1_ref, bu1_ref, bu2_ref, b2_ref,
             out_ref,
             x9_ref, merge_ref, acc_ref):

        def rolled(v, disp):
            s = (-disp) % HpWp
            return v if s == 0 else pltpu.roll(v, shift=s, axis=1)

        x2p = x2_ref[0]

        for t in range(9):
            ky, kx = t // 3, t % 3
            disp = (ky - 1) * Wp + (kx - 1)
            x9_ref[t * Ci:(t + 1) * Ci, :] = rolled(x2p, disp)
        a1 = jnp.dot(w1s_ref[...], x9_ref[...], preferred_element_type=f32)
        input1 = jnp.maximum(a1 + b1_ref[...], 0.0)
        input1_bf = input1.astype(bf16)

        g1 = jnp.dot(wu1s_ref[...], input1_bf, preferred_element_type=f32)
        g2 = jnp.dot(wu2s_ref[...], x2p, preferred_element_type=f32)

        mask = mask_ref[...]
        bu1 = bu1_ref[...]
        bu2 = bu2_ref[...]
        for py in range(2):
            for px in range(2):
                ph = py * 2 + px
                base = _GPOS[(py, px)] * Cm
                r1 = mask * jnp.maximum(g1[ph * Co:(ph + 1) * Co] + bu1, 0.0)
                merge_ref[base:base + Co, :] = r1.astype(bf16)
                r2 = mask * jnp.maximum(g2[ph * Co:(ph + 1) * Co] + bu2, 0.0)
                merge_ref[base + Co:base + 2 * Co, :] = r2.astype(bf16)
                merge_ref[base + 2 * Co:base + Cm, :] = (
                    x3_ref[0, ph * 32:(ph + 1) * 32, :])
        merge_ref[4 * Cm:5 * Cm, :] = merge_ref[0:Cm, :]

        accA = jnp.dot(wa_ref[...], merge_ref[0:4 * Cm, :],
                       preferred_element_type=f32)
        acc_ref[...] = accA

        for qx in range(2):
            k0 = (1 * Cm, 3 * Cm) if qx == 0 else (3 * Cm, 5 * Cm)
            t = jnp.dot(wb_ref[qx], merge_ref[k0[0]:k0[1], :],
                        preferred_element_type=f32)
            t = rolled(t, -1 if qx == 0 else 1)
            acc_ref[qx * Co:(qx + 1) * Co, :] += t[0:Co]
            acc_ref[(2 + qx) * Co:(3 + qx) * Co, :] += t[Co:2 * Co]

        for qy in range(2):
            k0 = (2 * Cm, 4 * Cm) if qy == 0 else (0, 2 * Cm)
            t = jnp.dot(wc_ref[qy], merge_ref[k0[0]:k0[1], :],
                        preferred_element_type=f32)
            t = rolled(t, -Wp if qy == 0 else Wp)
            acc_ref[(qy * 2) * Co:(qy * 2 + 1) * Co, :] += t[0:Co]
            acc_ref[(qy * 2 + 1) * Co:(qy * 2 + 2) * Co, :] += t[Co:2 * Co]

        for q in range(4):
            qy, qx = q // 2, q % 2
            pc = _GPOS[(1 - qy, 1 - qx)]
            t = jnp.dot(wcn_ref[q], merge_ref[pc * Cm:(pc + 1) * Cm, :],
                        preferred_element_type=f32)
            disp = (Wp if qy else -Wp) + (1 if qx else -1)
            acc_ref[q * Co:(q + 1) * Co, :] += rolled(t, disp)

        b2 = b2_ref[...]
        for q in range(4):
            o = jnp.maximum(acc_ref[q * Co:(q + 1) * Co, :] + b2, 0.0)
            out_ref[0, :, q * HWp:(q + 1) * HWp] = (
                o[:, Wp:Wp + HWp].astype(out_ref.dtype))

    return body


def _pack_conv2(w2, Co, Cm):
    planes = [(0, 0), (0, 1), (1, 1), (1, 0)]
    wa = jnp.concatenate([
        jnp.concatenate([w2[_K0[(qy, py)], _K0[(qx, px)]]
                         for (py, px) in planes], axis=1)
        for qy in range(2) for qx in range(2)], axis=0)
    wb = jnp.stack([
        jnp.concatenate([
            jnp.concatenate([w2[_K0[(qy, py)], 2 * qx]
                             for py in ([0, 1] if qx == 0 else [1, 0])],
                            axis=1)
            for qy in range(2)], axis=0)
        for qx in range(2)], axis=0)
    wc = jnp.stack([
        jnp.concatenate([
            jnp.concatenate([w2[2 * qy, _K0[(qx, px)]]
                             for px in ([1, 0] if qy == 0 else [0, 1])],
                            axis=1)
            for qx in range(2)], axis=0)
        for qy in range(2)], axis=0)
    wcn = jnp.stack([w2[2 * (q // 2), 2 * (q % 2)] for q in range(4)],
                    axis=0)
    return wa, wb, wc, wcn


def kernel(input2, input3, w1, b1, wu1, bu1, wu2, bu2, w2, b2):
    N, Ci, H, W = input2.shape
    Co = b1.shape[0]
    Hp, Wp = H + 2, W + 2
    HpWp = Hp * Wp
    HWp = H * Wp
    Cm = 2 * Co + 32
    bf16 = jnp.bfloat16
    f32 = jnp.float32

    x2p = jnp.pad(input2, ((0, 0), (0, 0), (1, 1), (1, 1)))
    x2p = x2p.reshape(N, Ci, HpWp).astype(bf16)
    x3 = input3.reshape(N, 32, H, 2, W, 2)
    x3 = jnp.transpose(x3, (0, 3, 5, 1, 2, 4)).reshape(N, 4, 32, H, W)
    x3 = jnp.pad(x3, ((0, 0), (0, 0), (0, 0), (1, 1), (1, 1)))
    x3 = x3.reshape(N, 128, HpWp).astype(bf16)

    mask = jnp.zeros((Hp, Wp), f32).at[1:Hp - 1, 1:Wp - 1].set(1.0)
    mask = mask.reshape(1, HpWp)

    w1s = jnp.concatenate([w1[t // 3, t % 3] for t in range(9)],
                          axis=1).astype(bf16)
    wu1s = jnp.concatenate([wu1[p // 2, p % 2] for p in range(4)],
                           axis=0).astype(bf16)
    wu2s = jnp.concatenate([wu2[p // 2, p % 2] for p in range(4)],
                           axis=0).astype(bf16)
    wa, wb, wc, wcn = _pack_conv2(w2, Co, Cm)
    wa, wb, wc, wcn = (x.astype(bf16) for x in (wa, wb, wc, wcn))

    body = _make_body(H, W, Ci, Co)
    full = lambda *s: pl.BlockSpec(s, lambda n: (0,) * len(s))

    out_flat = pl.pallas_call(
        body,
        out_shape=jax.ShapeDtypeStruct((N, Co, 4 * HWp), input2.dtype),
        grid=(N,),
        in_specs=[
            pl.BlockSpec((1, Ci, HpWp), lambda n: (n, 0, 0)),
            pl.BlockSpec((1, 128, HpWp), lambda n: (n, 0, 0)),
            full(1, HpWp),
            full(Co, 9 * Ci),
            full(4 * Co, Co),
            full(4 * Co, Ci),
            full(4 * Co, 4 * Cm),
            full(2, 2 * Co, 2 * Cm),
            full(2, 2 * Co, 2 * Cm),
            full(4, Co, Cm),
            full(Co, 1), full(Co, 1), full(Co, 1), full(Co, 1),
        ],
        out_specs=pl.BlockSpec((1, Co, 4 * HWp), lambda n: (n, 0, 0)),
        scratch_shapes=[
            pltpu.VMEM((9 * Ci, HpWp), bf16),
            pltpu.VMEM((5 * Cm, HpWp), bf16),
            pltpu.VMEM((4 * Co, HpWp), f32),
        ],
        compiler_params=pltpu.CompilerParams(
            dimension_semantics=("parallel",)),
    )(x2p, x3, mask, w1s, wu1s, wu2s, wa, wb, wc, wcn, b1, bu1, bu2, b2)

    y = out_flat.reshape(N, Co, 2, 2, H, Wp)[..., 1:1 + W]
    y = jnp.transpose(y, (0, 1, 4, 2, 5, 3)).reshape(N, Co, 2 * H, 2 * W)
    return y
